# baseline (speedup 1.0000x reference)
import jax
import jax.numpy as jnp
from jax.experimental import pallas as pl
from jax.experimental.pallas import tpu as pltpu

B, N, C, W, H = 8, 16, 64, 32, 32
WH = W * H
TS = 64


def _assemble_kernel(x_ref, t_ref, emb_ref, out_ref, tv_ref, sem):
    for n in range(N):
        row = emb_ref[t_ref[n], :]
        tv_ref[n] = jax.lax.broadcast_in_dim(row, (TS, WH), (0,))

    copies = []
    for b in range(B):
        copies.append(
            pltpu.make_async_copy(x_ref.at[b], out_ref.at[b, :, 0:C, :], sem))
        copies.append(
            pltpu.make_async_copy(tv_ref, out_ref.at[b, :, C:, :], sem))
    for c in copies:
        c.start()
    for c in copies:
        c.wait()


def kernel(x, t, emb_table):
    x2 = x.reshape(B, N, C, WH)
    out = pl.pallas_call(
        _assemble_kernel,
        in_specs=[
            pl.BlockSpec(memory_space=pl.ANY),
            pl.BlockSpec(memory_space=pltpu.SMEM),
            pl.BlockSpec(memory_space=pltpu.VMEM),
        ],
        out_specs=pl.BlockSpec(memory_space=pl.ANY),
        out_shape=jax.ShapeDtypeStruct((B, N, C + TS, WH), x.dtype),
        scratch_shapes=[
            pltpu.VMEM((N, TS, WH), x.dtype),
            pltpu.SemaphoreType.DMA,
        ],
    )(x2, t.astype(jnp.int32), emb_table)
    return out.reshape(B, N, C + TS, W, H)


# speedup vs baseline: 5.5950x; 5.5950x over previous
import jax
import jax.numpy as jnp
from jax.experimental import pallas as pl
from jax.experimental.pallas import tpu as pltpu

B, N, C, W, H = 8, 16, 64, 32, 32
WH = W * H
TS = 64


def _assemble_kernel(x_ref, t_ref, emb_ref, out_ref, tv_ref):
    i = pl.program_id(0)
    j = pl.program_id(1)

    @pl.when(jnp.logical_and(i == 0, j == 0))
    def _():
        for n in range(N):
            row = emb_ref[t_ref[n], :]
            tv_ref[n] = jax.lax.broadcast_in_dim(row, (TS, WH), (0,))

    out_ref[0, 0, :C, :] = x_ref[0, 0]
    out_ref[0, 0, C:, :] = tv_ref[j]


def kernel(x, t, emb_table):
    x2 = x.reshape(B, N, C, WH)
    out = pl.pallas_call(
        _assemble_kernel,
        grid=(B, N),
        in_specs=[
            pl.BlockSpec((1, 1, C, WH), lambda i, j: (i, j, 0, 0)),
            pl.BlockSpec(memory_space=pltpu.SMEM),
            pl.BlockSpec(memory_space=pltpu.VMEM),
        ],
        out_specs=pl.BlockSpec((1, 1, C + TS, WH), lambda i, j: (i, j, 0, 0)),
        out_shape=jax.ShapeDtypeStruct((B, N, C + TS, WH), x.dtype),
        scratch_shapes=[pltpu.VMEM((N, TS, WH), x.dtype)],
    )(x2, t.astype(jnp.int32), emb_table)
    return out.reshape(B, N, C + TS, W, H)


# speedup vs baseline: 8.0115x; 1.4319x over previous
import jax
import jax.numpy as jnp
from jax.experimental import pallas as pl
from jax.experimental.pallas import tpu as pltpu

B, N, C, W, H = 8, 16, 64, 32, 32
WH = W * H
TS = 64
CH = 32
ROWS = (B * N) // CH
NBUF = 16
NGRP = N // ROWS


def _assemble_kernel(x_ref, t_ref, emb_ref, out_ref, stage_ref, insem, outsem):
    for k in range(NBUF):
        for r in range(ROWS):
            n = (k % NGRP) * ROWS + r
            row = emb_ref[t_ref[n], :]
            stage_ref[k, r, C:, :] = jax.lax.broadcast_in_dim(row, (TS, WH), (0,))

    def in_copy(c):
        k = c % NBUF
        return pltpu.make_async_copy(
            x_ref.at[c], stage_ref.at[k, :, 0:C, :], insem.at[k])

    def out_copy(c):
        k = c % NBUF
        return pltpu.make_async_copy(stage_ref.at[k], out_ref.at[c], outsem.at[k])

    ins = {}
    outs = {}
    for c in range(NBUF):
        ins[c] = in_copy(c)
        ins[c].start()
    for c in range(NBUF):
        ins[c].wait()
        outs[c] = out_copy(c)
        outs[c].start()
    for c in range(NBUF, CH):
        outs[c - NBUF].wait()
        ins[c] = in_copy(c)
        ins[c].start()
    for c in range(NBUF, CH):
        ins[c].wait()
        outs[c] = out_copy(c)
        outs[c].start()
    for c in range(NBUF, CH):
        outs[c].wait()


def kernel(x, t, emb_table):
    x4 = x.reshape(CH, ROWS, C, WH)
    out = pl.pallas_call(
        _assemble_kernel,
        in_specs=[
            pl.BlockSpec(memory_space=pl.ANY),
            pl.BlockSpec(memory_space=pltpu.SMEM),
            pl.BlockSpec(memory_space=pltpu.VMEM),
        ],
        out_specs=pl.BlockSpec(memory_space=pl.ANY),
        out_shape=jax.ShapeDtypeStruct((CH, ROWS, C + TS, WH), x.dtype),
        scratch_shapes=[
            pltpu.VMEM((NBUF, ROWS, C + TS, WH), x.dtype),
            pltpu.SemaphoreType.DMA((NBUF,)),
            pltpu.SemaphoreType.DMA((NBUF,)),
        ],
    )(x4, t.astype(jnp.int32), emb_table)
    return out.reshape(B, N, C + TS, W, H)
